# baseline (device time: 107236 ns/iter reference)
import jax
import jax.numpy as jnp
from jax import lax
from jax.experimental import pallas as pl
from jax.experimental.pallas import tpu as pltpu

N_DEV = 4
SQ_SHARD = 256
SQ = 1024
D_MODEL = 1024
HQ_SHARD = 8
DH = 128
WINDOW = 128
KV_WIN = 512
KV_USED = 1152
SCALE = 0.08838834764831843


def _attn_body(x_ref, wq_ref, k_ref, v_ref, wo_ref, out_ref,
               xg_ref, q_ref, ctx_ref, p_ref, rs_ref,
               ag_send, ag_recv, rs_send, rs_recv):
    my = lax.axis_index("i")
    left = (my + N_DEV - 1) % N_DEV
    right = (my + 1) % N_DEV

    barrier_sem = pltpu.get_barrier_semaphore()
    for nbr in (left, right):
        pl.semaphore_signal(barrier_sem, inc=1, device_id=(nbr,),
                            device_id_type=pl.DeviceIdType.MESH)
    pl.semaphore_wait(barrier_sem, 2)

    xg_ref[pl.ds(my * SQ_SHARD, SQ_SHARD), :] = x_ref[:, :]
    for h in range(N_DEV - 1):
        si = (my + N_DEV - h) % N_DEV
        rdma = pltpu.make_async_remote_copy(
            src_ref=xg_ref.at[pl.ds(si * SQ_SHARD, SQ_SHARD), :],
            dst_ref=xg_ref.at[pl.ds(si * SQ_SHARD, SQ_SHARD), :],
            send_sem=ag_send.at[h],
            recv_sem=ag_recv.at[h],
            device_id=(right,),
            device_id_type=pl.DeviceIdType.MESH,
        )
        rdma.start()
        rdma.wait()

    q_ref[:, :] = jnp.dot(
        xg_ref[:, :], wq_ref[:, :],
        preferred_element_type=jnp.float32).astype(jnp.bfloat16)
    for c in range(N_DEV):
        ws = min(max(SQ_SHARD * c - WINDOW, 0), KV_USED - KV_WIN)
        iq = lax.broadcasted_iota(jnp.int32, (SQ_SHARD, KV_WIN), 0) \
            + SQ_SHARD * c
        ik = lax.broadcasted_iota(jnp.int32, (SQ_SHARD, KV_WIN), 1) + ws
        mask = jnp.abs(iq - ik) <= WINDOW
        for hh in range(HQ_SHARD):
            qh = q_ref[c * SQ_SHARD:(c + 1) * SQ_SHARD,
                       hh * DH:(hh + 1) * DH]
            kh = k_ref[hh, ws:ws + KV_WIN, :]
            s = lax.dot_general(
                qh, kh, (((1,), (1,)), ((), ())),
                preferred_element_type=jnp.float32) * SCALE
            s = jnp.where(mask, s, -1e9)
            m = jnp.max(s, axis=1, keepdims=True)
            w = jnp.exp(s - m)
            p = w / jnp.sum(w, axis=1, keepdims=True)
            vh = v_ref[hh, ws:ws + KV_WIN, :]
            ctx = jnp.dot(p.astype(jnp.bfloat16), vh,
                          preferred_element_type=jnp.float32)
            ctx_ref[:, hh * DH:(hh + 1) * DH] = ctx.astype(jnp.bfloat16)
        p_ref[c * SQ_SHARD:(c + 1) * SQ_SHARD, :] = jnp.dot(
            ctx_ref[:, :], wo_ref[:, :],
            preferred_element_type=jnp.float32)

    for t in range(N_DEV - 1):
        sc = (my + N_DEV - 1 - t) % N_DEV
        rc = (my + N_DEV - 2 - t) % N_DEV
        rdma = pltpu.make_async_remote_copy(
            src_ref=p_ref.at[pl.ds(sc * SQ_SHARD, SQ_SHARD), :],
            dst_ref=rs_ref.at[t],
            send_sem=rs_send.at[t],
            recv_sem=rs_recv.at[t],
            device_id=(right,),
            device_id_type=pl.DeviceIdType.MESH,
        )
        rdma.start()
        rdma.wait()
        p_ref[pl.ds(rc * SQ_SHARD, SQ_SHARD), :] = (
            p_ref[pl.ds(rc * SQ_SHARD, SQ_SHARD), :] + rs_ref[t, :, :])

    out_ref[:, :] = p_ref[pl.ds(my * SQ_SHARD, SQ_SHARD), :]


def kernel(x, Wq, K_ext, V_ext, Wo):
    i = lax.axis_index("i")
    xs = x[0].astype(jnp.bfloat16)
    wq = Wq.astype(jnp.bfloat16)
    wo = Wo.astype(jnp.bfloat16)
    k = lax.dynamic_slice_in_dim(
        K_ext[0, :KV_USED], i * HQ_SHARD, HQ_SHARD, axis=1)
    v = lax.dynamic_slice_in_dim(
        V_ext[0, :KV_USED], i * HQ_SHARD, HQ_SHARD, axis=1)
    k = jnp.transpose(k, (1, 0, 2)).astype(jnp.bfloat16)
    v = jnp.transpose(v, (1, 0, 2)).astype(jnp.bfloat16)

    out = pl.pallas_call(
        _attn_body,
        out_shape=jax.ShapeDtypeStruct((SQ_SHARD, D_MODEL), jnp.float32),
        in_specs=[pl.BlockSpec(memory_space=pltpu.VMEM)] * 5,
        out_specs=pl.BlockSpec(memory_space=pltpu.VMEM),
        scratch_shapes=[
            pltpu.VMEM((SQ, D_MODEL), jnp.bfloat16),
            pltpu.VMEM((SQ, D_MODEL), jnp.bfloat16),
            pltpu.VMEM((SQ_SHARD, D_MODEL), jnp.bfloat16),
            pltpu.VMEM((SQ, D_MODEL), jnp.float32),
            pltpu.VMEM((N_DEV - 1, SQ_SHARD, D_MODEL), jnp.float32),
            pltpu.SemaphoreType.DMA((N_DEV - 1,)),
            pltpu.SemaphoreType.DMA((N_DEV - 1,)),
            pltpu.SemaphoreType.DMA((N_DEV - 1,)),
            pltpu.SemaphoreType.DMA((N_DEV - 1,)),
        ],
        compiler_params=pltpu.CompilerParams(collective_id=0),
    )(xs, wq, k, v, wo)
    return out[None]


# device time: 65300 ns/iter; 1.6422x vs baseline; 1.6422x over previous
import jax
import jax.numpy as jnp
from jax import lax
from jax.experimental import pallas as pl
from jax.experimental.pallas import tpu as pltpu

N_DEV = 4
SQ_SHARD = 256
SQ = 1024
D_MODEL = 1024
HQ_SHARD = 8
DH = 128
WINDOW = 128
KV_WIN = 512
KV_USED = 1152
SCALE = 0.08838834764831843


def _attn_body(x_ref, wq_ref, k_ref, v_ref, wo_ref, out_ref,
               xg_ref, ctx_ref, p_ref, rs_tx, rs_rx,
               ag_send, ag_recv, rs_send, rs_recv):
    my = lax.axis_index("i")
    left = (my + N_DEV - 1) % N_DEV
    right = (my + 1) % N_DEV

    def compute_partial(c):
        qb = pl.multiple_of(c * SQ_SHARD, SQ_SHARD)
        ws = pl.multiple_of(
            jnp.clip(c * SQ_SHARD - WINDOW, 0, KV_USED - KV_WIN), WINDOW)
        qc = jnp.dot(xg_ref[pl.ds(qb, SQ_SHARD), :], wq_ref[:, :],
                     preferred_element_type=jnp.float32).astype(jnp.bfloat16)
        iq = lax.broadcasted_iota(jnp.int32, (SQ_SHARD, KV_WIN), 0) + qb
        ik = lax.broadcasted_iota(jnp.int32, (SQ_SHARD, KV_WIN), 1) + ws
        mask = jnp.abs(iq - ik) <= WINDOW
        for hh in range(HQ_SHARD):
            qh = qc[:, hh * DH:(hh + 1) * DH]
            kh = k_ref[hh, pl.ds(ws, KV_WIN), :]
            s = lax.dot_general(
                qh, kh, (((1,), (1,)), ((), ())),
                preferred_element_type=jnp.float32) * SCALE
            s = jnp.where(mask, s, -1e9)
            m = jnp.max(s, axis=1, keepdims=True)
            w = jnp.exp(s - m)
            p = w / jnp.sum(w, axis=1, keepdims=True)
            vh = v_ref[hh, pl.ds(ws, KV_WIN), :]
            ctx = jnp.dot(p.astype(jnp.bfloat16), vh,
                          preferred_element_type=jnp.float32)
            ctx_ref[:, hh * DH:(hh + 1) * DH] = ctx.astype(jnp.bfloat16)
        p_ref[pl.ds(qb, SQ_SHARD), :] = jnp.dot(
            ctx_ref[:, :], wo_ref[:, :], preferred_element_type=jnp.float32)

    def ag_hop(h, chunk):
        d = pltpu.make_async_remote_copy(
            src_ref=xg_ref.at[pl.ds(chunk * SQ_SHARD, SQ_SHARD), :],
            dst_ref=xg_ref.at[pl.ds(chunk * SQ_SHARD, SQ_SHARD), :],
            send_sem=ag_send.at[h],
            recv_sem=ag_recv.at[h],
            device_id=(right,),
            device_id_type=pl.DeviceIdType.MESH,
        )
        d.start()
        return d

    def rs_step(t):
        d = pltpu.make_async_remote_copy(
            src_ref=rs_tx.at[t],
            dst_ref=rs_rx.at[t],
            send_sem=rs_send.at[t],
            recv_sem=rs_recv.at[t],
            device_id=(right,),
            device_id_type=pl.DeviceIdType.MESH,
        )
        d.start()
        return d

    barrier_sem = pltpu.get_barrier_semaphore()
    for nbr in (left, right):
        pl.semaphore_signal(barrier_sem, inc=1, device_id=(nbr,),
                            device_id_type=pl.DeviceIdType.MESH)
    pl.semaphore_wait(barrier_sem, 2)

    xg_ref[pl.ds(my * SQ_SHARD, SQ_SHARD), :] = x_ref[:, :]
    ag = [ag_hop(0, my)]
    compute_partial(my)

    rs = []
    for h in range(N_DEV - 1):
        ag[h].wait_recv()
        c = (my + N_DEV - 1 - h) % N_DEV
        if h < N_DEV - 2:
            ag.append(ag_hop(h + 1, c))
        compute_partial(c)
        if h == 0:
            acc = p_ref[pl.ds(c * SQ_SHARD, SQ_SHARD), :]
        else:
            rs[h - 1].wait_recv()
            acc = (p_ref[pl.ds(c * SQ_SHARD, SQ_SHARD), :]
                   + rs_rx[h - 1, :, :].astype(jnp.float32))
        rs_tx[h, :, :] = acc.astype(jnp.bfloat16)
        rs.append(rs_step(h))

    rs[N_DEV - 2].wait_recv()
    out_ref[:, :] = (p_ref[pl.ds(my * SQ_SHARD, SQ_SHARD), :]
                     + rs_rx[N_DEV - 2, :, :].astype(jnp.float32))

    for d in ag + rs:
        d.wait_send()


def kernel(x, Wq, K_ext, V_ext, Wo):
    i = lax.axis_index("i")
    xs = x[0].astype(jnp.bfloat16)
    wq = Wq.astype(jnp.bfloat16)
    wo = Wo.astype(jnp.bfloat16)
    k = lax.dynamic_slice_in_dim(
        K_ext[0, :KV_USED], i * HQ_SHARD, HQ_SHARD, axis=1)
    v = lax.dynamic_slice_in_dim(
        V_ext[0, :KV_USED], i * HQ_SHARD, HQ_SHARD, axis=1)
    k = jnp.transpose(k, (1, 0, 2)).astype(jnp.bfloat16)
    v = jnp.transpose(v, (1, 0, 2)).astype(jnp.bfloat16)

    out = pl.pallas_call(
        _attn_body,
        out_shape=jax.ShapeDtypeStruct((SQ_SHARD, D_MODEL), jnp.float32),
        in_specs=[pl.BlockSpec(memory_space=pltpu.VMEM)] * 5,
        out_specs=pl.BlockSpec(memory_space=pltpu.VMEM),
        scratch_shapes=[
            pltpu.VMEM((SQ, D_MODEL), jnp.bfloat16),
            pltpu.VMEM((SQ_SHARD, D_MODEL), jnp.bfloat16),
            pltpu.VMEM((SQ, D_MODEL), jnp.float32),
            pltpu.VMEM((N_DEV - 1, SQ_SHARD, D_MODEL), jnp.bfloat16),
            pltpu.VMEM((N_DEV - 1, SQ_SHARD, D_MODEL), jnp.bfloat16),
            pltpu.SemaphoreType.DMA((N_DEV - 1,)),
            pltpu.SemaphoreType.DMA((N_DEV - 1,)),
            pltpu.SemaphoreType.DMA((N_DEV - 1,)),
            pltpu.SemaphoreType.DMA((N_DEV - 1,)),
        ],
        compiler_params=pltpu.CompilerParams(collective_id=0),
    )(xs, wq, k, v, wo)
    return out[None]


# device time: 56105 ns/iter; 1.9113x vs baseline; 1.1639x over previous
import jax
import jax.numpy as jnp
from jax import lax
from jax.experimental import pallas as pl
from jax.experimental.pallas import tpu as pltpu

N_DEV = 4
SQ_SHARD = 256
SQ = 1024
D_MODEL = 1024
HQ_SHARD = 8
DH = 128
WINDOW = 128
KV_WIN = 512
KV_USED = 1152
SCALE = 0.08838834764831843


def _attn_body(x_ref, wq_ref, k_ref, v_ref, wo_ref, out_ref,
               xg_ref, ctx_ref, p_ref, rs_tx, rs_rx,
               ag_send, ag_recv, rs_send, rs_recv):
    my = lax.axis_index("i")
    left = (my + N_DEV - 1) % N_DEV
    right = (my + 1) % N_DEV

    def compute_partial(c):
        qb = pl.multiple_of(c * SQ_SHARD, SQ_SHARD)
        ws = pl.multiple_of(
            jnp.clip(c * SQ_SHARD - WINDOW, 0, KV_USED - KV_WIN), WINDOW)
        qc = (jnp.dot(xg_ref[pl.ds(qb, SQ_SHARD), :], wq_ref[:, :],
                      preferred_element_type=jnp.float32)
              * SCALE).astype(jnp.bfloat16)
        iq = lax.broadcasted_iota(jnp.int32, (SQ_SHARD, KV_WIN), 0) + qb
        ik = lax.broadcasted_iota(jnp.int32, (SQ_SHARD, KV_WIN), 1) + ws
        bias = jnp.where(jnp.abs(iq - ik) <= WINDOW,
                         0.0, -1e9).astype(jnp.float32)
        for hh in range(HQ_SHARD):
            qh = qc[:, hh * DH:(hh + 1) * DH]
            kh = k_ref[hh, pl.ds(ws, KV_WIN), :]
            s = lax.dot_general(
                qh, kh, (((1,), (1,)), ((), ())),
                preferred_element_type=jnp.float32) + bias
            w = jnp.exp(s)
            denom = jnp.sum(w, axis=1, keepdims=True)
            vh = v_ref[hh, pl.ds(ws, KV_WIN), :]
            ctx = jnp.dot(w.astype(jnp.bfloat16), vh,
                          preferred_element_type=jnp.float32)
            ctx = ctx * (1.0 / denom)
            ctx_ref[:, hh * DH:(hh + 1) * DH] = ctx.astype(jnp.bfloat16)
        p_ref[pl.ds(qb, SQ_SHARD), :] = jnp.dot(
            ctx_ref[:, :], wo_ref[:, :], preferred_element_type=jnp.float32)

    def ag_hop(h, chunk):
        d = pltpu.make_async_remote_copy(
            src_ref=xg_ref.at[pl.ds(chunk * SQ_SHARD, SQ_SHARD), :],
            dst_ref=xg_ref.at[pl.ds(chunk * SQ_SHARD, SQ_SHARD), :],
            send_sem=ag_send.at[h],
            recv_sem=ag_recv.at[h],
            device_id=(right,),
            device_id_type=pl.DeviceIdType.MESH,
        )
        d.start()
        return d

    def rs_step(t):
        d = pltpu.make_async_remote_copy(
            src_ref=rs_tx.at[t],
            dst_ref=rs_rx.at[t],
            send_sem=rs_send.at[t],
            recv_sem=rs_recv.at[t],
            device_id=(right,),
            device_id_type=pl.DeviceIdType.MESH,
        )
        d.start()
        return d

    barrier_sem = pltpu.get_barrier_semaphore()
    for nbr in (left, right):
        pl.semaphore_signal(barrier_sem, inc=1, device_id=(nbr,),
                            device_id_type=pl.DeviceIdType.MESH)
    pl.semaphore_wait(barrier_sem, 2)

    xg_ref[pl.ds(my * SQ_SHARD, SQ_SHARD), :] = x_ref[:, :]
    ag = [ag_hop(0, my)]
    compute_partial(my)

    rs = []
    for h in range(N_DEV - 1):
        ag[h].wait_recv()
        c = (my + N_DEV - 1 - h) % N_DEV
        if h < N_DEV - 2:
            ag.append(ag_hop(h + 1, c))
        compute_partial(c)
        if h == 0:
            acc = p_ref[pl.ds(c * SQ_SHARD, SQ_SHARD), :]
        else:
            rs[h - 1].wait_recv()
            acc = (p_ref[pl.ds(c * SQ_SHARD, SQ_SHARD), :]
                   + rs_rx[h - 1, :, :].astype(jnp.float32))
        rs_tx[h, :, :] = acc.astype(jnp.bfloat16)
        rs.append(rs_step(h))

    rs[N_DEV - 2].wait_recv()
    out_ref[0, :, :] = (p_ref[pl.ds(my * SQ_SHARD, SQ_SHARD), :]
                        + rs_rx[N_DEV - 2, :, :].astype(jnp.float32))

    for d in ag + rs:
        d.wait_send()


def kernel(x, Wq, K_ext, V_ext, Wo):
    i = lax.axis_index("i")
    xs = x[0].astype(jnp.bfloat16)
    wq = Wq.astype(jnp.bfloat16)
    wo = Wo.astype(jnp.bfloat16)
    k = lax.dynamic_slice(
        K_ext, (0, 0, i * HQ_SHARD, 0), (1, KV_USED, HQ_SHARD, DH))[0]
    v = lax.dynamic_slice(
        V_ext, (0, 0, i * HQ_SHARD, 0), (1, KV_USED, HQ_SHARD, DH))[0]
    k = jnp.transpose(k, (1, 0, 2)).astype(jnp.bfloat16)
    v = jnp.transpose(v, (1, 0, 2)).astype(jnp.bfloat16)

    out = pl.pallas_call(
        _attn_body,
        out_shape=jax.ShapeDtypeStruct((1, SQ_SHARD, D_MODEL), jnp.float32),
        in_specs=[pl.BlockSpec(memory_space=pltpu.VMEM)] * 5,
        out_specs=pl.BlockSpec(memory_space=pltpu.VMEM),
        scratch_shapes=[
            pltpu.VMEM((SQ, D_MODEL), jnp.bfloat16),
            pltpu.VMEM((SQ_SHARD, D_MODEL), jnp.bfloat16),
            pltpu.VMEM((SQ, D_MODEL), jnp.float32),
            pltpu.VMEM((N_DEV - 1, SQ_SHARD, D_MODEL), jnp.bfloat16),
            pltpu.VMEM((N_DEV - 1, SQ_SHARD, D_MODEL), jnp.bfloat16),
            pltpu.SemaphoreType.DMA((N_DEV - 1,)),
            pltpu.SemaphoreType.DMA((N_DEV - 1,)),
            pltpu.SemaphoreType.DMA((N_DEV - 1,)),
            pltpu.SemaphoreType.DMA((N_DEV - 1,)),
        ],
        compiler_params=pltpu.CompilerParams(collective_id=0),
    )(xs, wq, k, v, wo)
    return out


# device time: 43611 ns/iter; 2.4589x vs baseline; 1.2865x over previous
import jax
import jax.numpy as jnp
from jax import lax
from jax.experimental import pallas as pl
from jax.experimental.pallas import tpu as pltpu

N_DEV = 4
SQ_SHARD = 256
HALF = 128
SQ = 1024
D_MODEL = 1024
HQ_SHARD = 8
DH = 128
WINDOW = 128
KV_HWIN = 384
KV_USED = 1152
SCALE = 0.08838834764831843


def _attn_body(x_ref, wq_ref, k_ref, v_ref, wo_ref, out_ref,
               xg_ref, ctx_ref, p_ref,
               rs_txr, rs_rxr, rs_txl, rs_rxl,
               agr_send, agr_recv, agl_send, agl_recv,
               rsr_send, rsr_recv, rsl_send, rsl_recv):
    my = lax.axis_index("i")
    left = (my + N_DEV - 1) % N_DEV
    right = (my + 1) % N_DEV

    def compute_half(c, half):
        qb = pl.multiple_of(c * SQ_SHARD + half * HALF, HALF)
        ws = pl.multiple_of(
            jnp.clip(c * SQ_SHARD + half * HALF - WINDOW,
                     0, KV_USED - KV_HWIN), WINDOW)
        qc = (jnp.dot(xg_ref[pl.ds(qb, HALF), :], wq_ref[:, :],
                      preferred_element_type=jnp.float32)
              * SCALE).astype(jnp.bfloat16)
        iq = lax.broadcasted_iota(jnp.int32, (HALF, KV_HWIN), 0) + qb
        ik = lax.broadcasted_iota(jnp.int32, (HALF, KV_HWIN), 1) + ws
        bias = jnp.where(jnp.abs(iq - ik) <= WINDOW,
                         0.0, -1e9).astype(jnp.float32)
        for hh in range(HQ_SHARD):
            qh = qc[:, hh * DH:(hh + 1) * DH]
            kh = k_ref[hh, pl.ds(ws, KV_HWIN), :]
            s = lax.dot_general(
                qh, kh, (((1,), (1,)), ((), ())),
                preferred_element_type=jnp.float32) + bias
            w = jnp.exp(s)
            denom = jnp.sum(w, axis=1, keepdims=True)
            vh = v_ref[hh, pl.ds(ws, KV_HWIN), :]
            ctx = jnp.dot(w.astype(jnp.bfloat16), vh,
                          preferred_element_type=jnp.float32)
            ctx = ctx * (1.0 / denom)
            ctx_ref[:, hh * DH:(hh + 1) * DH] = ctx.astype(jnp.bfloat16)
        p_ref[pl.ds(qb, HALF), :] = jnp.dot(
            ctx_ref[:, :], wo_ref[:, :], preferred_element_type=jnp.float32)

    def ag_hop(h, c, half, dst, send_sems, recv_sems):
        rows = pl.ds(pl.multiple_of(c * SQ_SHARD + half * HALF, HALF), HALF)
        d = pltpu.make_async_remote_copy(
            src_ref=xg_ref.at[rows, :],
            dst_ref=xg_ref.at[rows, :],
            send_sem=send_sems.at[h],
            recv_sem=recv_sems.at[h],
            device_id=(dst,),
            device_id_type=pl.DeviceIdType.MESH,
        )
        d.start()
        return d

    def rs_step(t, tx, rx, dst, send_sems, recv_sems):
        d = pltpu.make_async_remote_copy(
            src_ref=tx.at[t],
            dst_ref=rx.at[t],
            send_sem=send_sems.at[t],
            recv_sem=recv_sems.at[t],
            device_id=(dst,),
            device_id_type=pl.DeviceIdType.MESH,
        )
        d.start()
        return d

    barrier_sem = pltpu.get_barrier_semaphore()
    for nbr in (left, right):
        pl.semaphore_signal(barrier_sem, inc=1, device_id=(nbr,),
                            device_id_type=pl.DeviceIdType.MESH)
    pl.semaphore_wait(barrier_sem, 2)

    xg_ref[pl.ds(pl.multiple_of(my * SQ_SHARD, SQ_SHARD), SQ_SHARD), :] = \
        x_ref[:, :]
    agr = [ag_hop(0, my, 0, right, agr_send, agr_recv)]
    agl = [ag_hop(0, my, 1, left, agl_send, agl_recv)]
    compute_half(my, 0)
    compute_half(my, 1)

    rsr, rsl = [], []
    for h in range(N_DEV - 1):
        agr[h].wait_recv()
        cr = (my + N_DEV - 1 - h) % N_DEV
        if h < N_DEV - 2:
            agr.append(ag_hop(h + 1, cr, 0, right, agr_send, agr_recv))
        agl[h].wait_recv()
        cl = (my + 1 + h) % N_DEV
        if h < N_DEV - 2:
            agl.append(ag_hop(h + 1, cl, 1, left, agl_send, agl_recv))

        compute_half(cr, 0)
        rows_r = pl.ds(pl.multiple_of(cr * SQ_SHARD, HALF), HALF)
        if h == 0:
            acc_r = p_ref[rows_r, :]
        else:
            rsr[h - 1].wait_recv()
            acc_r = p_ref[rows_r, :] + rs_rxr[h - 1, :, :].astype(jnp.float32)
        rs_txr[h, :, :] = acc_r.astype(jnp.bfloat16)
        rsr.append(rs_step(h, rs_txr, rs_rxr, right, rsr_send, rsr_recv))

        compute_half(cl, 1)
        rows_l = pl.ds(pl.multiple_of(cl * SQ_SHARD + HALF, HALF), HALF)
        if h == 0:
            acc_l = p_ref[rows_l, :]
        else:
            rsl[h - 1].wait_recv()
            acc_l = p_ref[rows_l, :] + rs_rxl[h - 1, :, :].astype(jnp.float32)
        rs_txl[h, :, :] = acc_l.astype(jnp.bfloat16)
        rsl.append(rs_step(h, rs_txl, rs_rxl, left, rsl_send, rsl_recv))

    rsr[N_DEV - 2].wait_recv()
    out_ref[0, 0:HALF, :] = (
        p_ref[pl.ds(pl.multiple_of(my * SQ_SHARD, HALF), HALF), :]
        + rs_rxr[N_DEV - 2, :, :].astype(jnp.float32))
    rsl[N_DEV - 2].wait_recv()
    out_ref[0, HALF:SQ_SHARD, :] = (
        p_ref[pl.ds(pl.multiple_of(my * SQ_SHARD + HALF, HALF), HALF), :]
        + rs_rxl[N_DEV - 2, :, :].astype(jnp.float32))

    for d in agr + agl + rsr + rsl:
        d.wait_send()


def kernel(x, Wq, K_ext, V_ext, Wo):
    i = lax.axis_index("i")
    xs = x[0].astype(jnp.bfloat16)
    wq = Wq.astype(jnp.bfloat16)
    wo = Wo.astype(jnp.bfloat16)
    k = lax.dynamic_slice(
        K_ext, (0, 0, i * HQ_SHARD, 0), (1, KV_USED, HQ_SHARD, DH))[0]
    v = lax.dynamic_slice(
        V_ext, (0, 0, i * HQ_SHARD, 0), (1, KV_USED, HQ_SHARD, DH))[0]
    k = jnp.transpose(k, (1, 0, 2)).astype(jnp.bfloat16)
    v = jnp.transpose(v, (1, 0, 2)).astype(jnp.bfloat16)

    out = pl.pallas_call(
        _attn_body,
        out_shape=jax.ShapeDtypeStruct((1, SQ_SHARD, D_MODEL), jnp.float32),
        in_specs=[pl.BlockSpec(memory_space=pltpu.VMEM)] * 5,
        out_specs=pl.BlockSpec(memory_space=pltpu.VMEM),
        scratch_shapes=[
            pltpu.VMEM((SQ, D_MODEL), jnp.bfloat16),
            pltpu.VMEM((HALF, D_MODEL), jnp.bfloat16),
            pltpu.VMEM((SQ, D_MODEL), jnp.float32),
            pltpu.VMEM((N_DEV - 1, HALF, D_MODEL), jnp.bfloat16),
            pltpu.VMEM((N_DEV - 1, HALF, D_MODEL), jnp.bfloat16),
            pltpu.VMEM((N_DEV - 1, HALF, D_MODEL), jnp.bfloat16),
            pltpu.VMEM((N_DEV - 1, HALF, D_MODEL), jnp.bfloat16),
            pltpu.SemaphoreType.DMA((N_DEV - 1,)),
            pltpu.SemaphoreType.DMA((N_DEV - 1,)),
            pltpu.SemaphoreType.DMA((N_DEV - 1,)),
            pltpu.SemaphoreType.DMA((N_DEV - 1,)),
            pltpu.SemaphoreType.DMA((N_DEV - 1,)),
            pltpu.SemaphoreType.DMA((N_DEV - 1,)),
            pltpu.SemaphoreType.DMA((N_DEV - 1,)),
            pltpu.SemaphoreType.DMA((N_DEV - 1,)),
        ],
        compiler_params=pltpu.CompilerParams(collective_id=0),
    )(xs, wq, k, v, wo)
    return out


# device time: 38364 ns/iter; 2.7952x vs baseline; 1.1368x over previous
import jax
import jax.numpy as jnp
from jax import lax
from jax.experimental import pallas as pl
from jax.experimental.pallas import tpu as pltpu

N_DEV = 4
SQ_SHARD = 256
HALF = 128
SQ = 1024
D_MODEL = 1024
HQ_SHARD = 8
DH = 128
WINDOW = 128
KV_HWIN = 384
KV_USED = 1152
SCALE = 0.08838834764831843


def _attn_body(x_ref, wq_ref, k_hbm, v_hbm, wo_ref, out_ref,
               xg_ref, ctx_ref, p_ref, k_ref, v_ref,
               rs_txr, rs_rxr, rs_txl, rs_rxl,
               kv_sems,
               agr_send, agr_recv, agl_send, agl_recv,
               rsr_send, rsr_recv, rsl_send, rsl_recv):
    my = lax.axis_index("i")
    left = (my + N_DEV - 1) % N_DEV
    right = (my + 1) % N_DEV

    h0 = my * HQ_SHARD
    kv_dmas = []
    for hh in range(HQ_SHARD):
        for src, dst, slot in ((k_hbm, k_ref, 2 * hh),
                               (v_hbm, v_ref, 2 * hh + 1)):
            d = pltpu.make_async_copy(
                src.at[0, pl.ds(0, KV_USED), h0 + hh, :],
                dst.at[hh],
                kv_sems.at[slot],
            )
            d.start()
            kv_dmas.append(d)

    def compute_half(c, half):
        qb = pl.multiple_of(c * SQ_SHARD + half * HALF, HALF)
        ws = pl.multiple_of(
            jnp.clip(c * SQ_SHARD + half * HALF - WINDOW,
                     0, KV_USED - KV_HWIN), WINDOW)
        qc = (jnp.dot(xg_ref[pl.ds(qb, HALF), :], wq_ref[:, :],
                      preferred_element_type=jnp.float32)
              * SCALE).astype(jnp.bfloat16)
        iq = lax.broadcasted_iota(jnp.int32, (HALF, KV_HWIN), 0) + qb
        ik = lax.broadcasted_iota(jnp.int32, (HALF, KV_HWIN), 1) + ws
        bias = jnp.where(jnp.abs(iq - ik) <= WINDOW,
                         0.0, -1e9).astype(jnp.float32)
        for hh in range(HQ_SHARD):
            qh = qc[:, hh * DH:(hh + 1) * DH]
            kh = k_ref[hh, pl.ds(ws, KV_HWIN), :].astype(jnp.bfloat16)
            s = lax.dot_general(
                qh, kh, (((1,), (1,)), ((), ())),
                preferred_element_type=jnp.float32) + bias
            w = jnp.exp(s)
            denom = jnp.sum(w, axis=1, keepdims=True)
            vh = v_ref[hh, pl.ds(ws, KV_HWIN), :].astype(jnp.bfloat16)
            ctx = jnp.dot(w.astype(jnp.bfloat16), vh,
                          preferred_element_type=jnp.float32)
            ctx = ctx * (1.0 / denom)
            ctx_ref[:, hh * DH:(hh + 1) * DH] = ctx.astype(jnp.bfloat16)
        p_ref[pl.ds(qb, HALF), :] = jnp.dot(
            ctx_ref[:, :], wo_ref[:, :], preferred_element_type=jnp.float32)

    def ag_hop(h, c, half, dst, send_sems, recv_sems):
        rows = pl.ds(pl.multiple_of(c * SQ_SHARD + half * HALF, HALF), HALF)
        d = pltpu.make_async_remote_copy(
            src_ref=xg_ref.at[rows, :],
            dst_ref=xg_ref.at[rows, :],
            send_sem=send_sems.at[h],
            recv_sem=recv_sems.at[h],
            device_id=(dst,),
            device_id_type=pl.DeviceIdType.MESH,
        )
        d.start()
        return d

    def rs_step(t, tx, rx, dst, send_sems, recv_sems):
        d = pltpu.make_async_remote_copy(
            src_ref=tx.at[t],
            dst_ref=rx.at[t],
            send_sem=send_sems.at[t],
            recv_sem=recv_sems.at[t],
            device_id=(dst,),
            device_id_type=pl.DeviceIdType.MESH,
        )
        d.start()
        return d

    barrier_sem = pltpu.get_barrier_semaphore()
    for nbr in (left, right):
        pl.semaphore_signal(barrier_sem, inc=1, device_id=(nbr,),
                            device_id_type=pl.DeviceIdType.MESH)
    pl.semaphore_wait(barrier_sem, 2)

    xg_ref[pl.ds(pl.multiple_of(my * SQ_SHARD, SQ_SHARD), SQ_SHARD), :] = \
        x_ref[:, :]
    agr = [ag_hop(0, my, 0, right, agr_send, agr_recv)]
    agl = [ag_hop(0, my, 1, left, agl_send, agl_recv)]
    for d in kv_dmas:
        d.wait()
    compute_half(my, 0)
    compute_half(my, 1)

    rsr, rsl = [], []
    for h in range(N_DEV - 1):
        agr[h].wait_recv()
        cr = (my + N_DEV - 1 - h) % N_DEV
        if h < N_DEV - 2:
            agr.append(ag_hop(h + 1, cr, 0, right, agr_send, agr_recv))
        agl[h].wait_recv()
        cl = (my + 1 + h) % N_DEV
        if h < N_DEV - 2:
            agl.append(ag_hop(h + 1, cl, 1, left, agl_send, agl_recv))

        compute_half(cr, 0)
        rows_r = pl.ds(pl.multiple_of(cr * SQ_SHARD, HALF), HALF)
        if h == 0:
            acc_r = p_ref[rows_r, :]
        else:
            rsr[h - 1].wait_recv()
            acc_r = p_ref[rows_r, :] + rs_rxr[h - 1, :, :].astype(jnp.float32)
        rs_txr[h, :, :] = acc_r.astype(jnp.bfloat16)
        rsr.append(rs_step(h, rs_txr, rs_rxr, right, rsr_send, rsr_recv))

        compute_half(cl, 1)
        rows_l = pl.ds(pl.multiple_of(cl * SQ_SHARD + HALF, HALF), HALF)
        if h == 0:
            acc_l = p_ref[rows_l, :]
        else:
            rsl[h - 1].wait_recv()
            acc_l = p_ref[rows_l, :] + rs_rxl[h - 1, :, :].astype(jnp.float32)
        rs_txl[h, :, :] = acc_l.astype(jnp.bfloat16)
        rsl.append(rs_step(h, rs_txl, rs_rxl, left, rsl_send, rsl_recv))

    rsr[N_DEV - 2].wait_recv()
    out_ref[0, 0:HALF, :] = (
        p_ref[pl.ds(pl.multiple_of(my * SQ_SHARD, HALF), HALF), :]
        + rs_rxr[N_DEV - 2, :, :].astype(jnp.float32))
    rsl[N_DEV - 2].wait_recv()
    out_ref[0, HALF:SQ_SHARD, :] = (
        p_ref[pl.ds(pl.multiple_of(my * SQ_SHARD + HALF, HALF), HALF), :]
        + rs_rxl[N_DEV - 2, :, :].astype(jnp.float32))

    for d in agr + agl + rsr + rsl:
        d.wait_send()


def kernel(x, Wq, K_ext, V_ext, Wo):
    xs = x[0].astype(jnp.bfloat16)
    wq = Wq.astype(jnp.bfloat16)
    wo = Wo.astype(jnp.bfloat16)

    out = pl.pallas_call(
        _attn_body,
        out_shape=jax.ShapeDtypeStruct((1, SQ_SHARD, D_MODEL), jnp.float32),
        in_specs=[
            pl.BlockSpec(memory_space=pltpu.VMEM),
            pl.BlockSpec(memory_space=pltpu.VMEM),
            pl.BlockSpec(memory_space=pltpu.MemorySpace.HBM),
            pl.BlockSpec(memory_space=pltpu.MemorySpace.HBM),
            pl.BlockSpec(memory_space=pltpu.VMEM),
        ],
        out_specs=pl.BlockSpec(memory_space=pltpu.VMEM),
        scratch_shapes=[
            pltpu.VMEM((SQ, D_MODEL), jnp.bfloat16),
            pltpu.VMEM((HALF, D_MODEL), jnp.bfloat16),
            pltpu.VMEM((SQ, D_MODEL), jnp.float32),
            pltpu.VMEM((HQ_SHARD, KV_USED, DH), jnp.float32),
            pltpu.VMEM((HQ_SHARD, KV_USED, DH), jnp.float32),
            pltpu.VMEM((N_DEV - 1, HALF, D_MODEL), jnp.bfloat16),
            pltpu.VMEM((N_DEV - 1, HALF, D_MODEL), jnp.bfloat16),
            pltpu.VMEM((N_DEV - 1, HALF, D_MODEL), jnp.bfloat16),
            pltpu.VMEM((N_DEV - 1, HALF, D_MODEL), jnp.bfloat16),
            pltpu.SemaphoreType.DMA((2 * HQ_SHARD,)),
            pltpu.SemaphoreType.DMA((N_DEV - 1,)),
            pltpu.SemaphoreType.DMA((N_DEV - 1,)),
            pltpu.SemaphoreType.DMA((N_DEV - 1,)),
            pltpu.SemaphoreType.DMA((N_DEV - 1,)),
            pltpu.SemaphoreType.DMA((N_DEV - 1,)),
            pltpu.SemaphoreType.DMA((N_DEV - 1,)),
            pltpu.SemaphoreType.DMA((N_DEV - 1,)),
            pltpu.SemaphoreType.DMA((N_DEV - 1,)),
        ],
        compiler_params=pltpu.CompilerParams(collective_id=0),
    )(xs, wq, K_ext, V_ext, wo)
    return out


# device time: 36121 ns/iter; 2.9688x vs baseline; 1.0621x over previous
import jax
import jax.numpy as jnp
from jax import lax
from jax.experimental import pallas as pl
from jax.experimental.pallas import tpu as pltpu

N_DEV = 4
SQ_SHARD = 256
HALF = 128
SQ = 1024
D_MODEL = 1024
HQ_SHARD = 8
DH = 128
WINDOW = 128
KV_HWIN = 384
KV_USED = 1152
SCALE = 0.08838834764831843


def _attn_body(x_ref, wq_ref, k_hbm, v_hbm, wo_ref, out_ref,
               xg_ref, xq_ref, ctx_ref, p_ref, k_ref, v_ref,
               rs_txr, rs_rxr, rs_txl, rs_rxl,
               kv_sems,
               agr_send, agr_recv, agl_send, agl_recv,
               rsr_send, rsr_recv, rsl_send, rsl_recv):
    my = lax.axis_index("i")
    left = (my + N_DEV - 1) % N_DEV
    right = (my + 1) % N_DEV

    h0 = my * HQ_SHARD
    kv_dmas = []
    for hh in range(HQ_SHARD):
        for src, dst, slot in ((k_hbm, k_ref, 2 * hh),
                               (v_hbm, v_ref, 2 * hh + 1)):
            d = pltpu.make_async_copy(
                src.at[0, pl.ds(0, KV_USED), h0 + hh, :],
                dst.at[hh],
                kv_sems.at[slot],
            )
            d.start()
            kv_dmas.append(d)
    kv_waited = [False] * HQ_SHARD

    def compute_pair(cr, cl):
        qb_r = pl.multiple_of(cr * SQ_SHARD, HALF)
        qb_l = pl.multiple_of(cl * SQ_SHARD + HALF, HALF)
        xq_ref[0:HALF, :] = xg_ref[pl.ds(qb_r, HALF), :]
        xq_ref[HALF:SQ_SHARD, :] = xg_ref[pl.ds(qb_l, HALF), :]
        qc = (jnp.dot(xq_ref[:, :], wq_ref[:, :],
                      preferred_element_type=jnp.float32)
              * SCALE).astype(jnp.bfloat16)
        for idx, qb in ((0, qb_r), (1, qb_l)):
            ws = pl.multiple_of(
                jnp.clip(qb - WINDOW, 0, KV_USED - KV_HWIN), WINDOW)
            iq = lax.broadcasted_iota(jnp.int32, (HALF, KV_HWIN), 0) + qb
            ik = lax.broadcasted_iota(jnp.int32, (HALF, KV_HWIN), 1) + ws
            bias = jnp.where(jnp.abs(iq - ik) <= WINDOW,
                             0.0, -1e9).astype(jnp.float32)
            r0 = idx * HALF
            for hh in range(HQ_SHARD):
                if not kv_waited[hh]:
                    kv_dmas[2 * hh].wait()
                    kv_dmas[2 * hh + 1].wait()
                    kv_waited[hh] = True
                qh = qc[r0:r0 + HALF, hh * DH:(hh + 1) * DH]
                kh = k_ref[hh, pl.ds(ws, KV_HWIN), :].astype(jnp.bfloat16)
                s = lax.dot_general(
                    qh, kh, (((1,), (1,)), ((), ())),
                    preferred_element_type=jnp.float32) + bias
                w = jnp.exp(s)
                denom = jnp.sum(w, axis=1, keepdims=True)
                vh = v_ref[hh, pl.ds(ws, KV_HWIN), :].astype(jnp.bfloat16)
                ctx = jnp.dot(w.astype(jnp.bfloat16), vh,
                              preferred_element_type=jnp.float32)
                ctx = ctx * (1.0 / denom)
                ctx_ref[r0:r0 + HALF,
                        hh * DH:(hh + 1) * DH] = ctx.astype(jnp.bfloat16)
        o = jnp.dot(ctx_ref[:, :], wo_ref[:, :],
                    preferred_element_type=jnp.float32)
        p_ref[pl.ds(qb_r, HALF), :] = o[0:HALF, :]
        p_ref[pl.ds(qb_l, HALF), :] = o[HALF:SQ_SHARD, :]

    def ag_hop(h, c, half, dst, send_sems, recv_sems):
        rows = pl.ds(pl.multiple_of(c * SQ_SHARD + half * HALF, HALF), HALF)
        d = pltpu.make_async_remote_copy(
            src_ref=xg_ref.at[rows, :],
            dst_ref=xg_ref.at[rows, :],
            send_sem=send_sems.at[h],
            recv_sem=recv_sems.at[h],
            device_id=(dst,),
            device_id_type=pl.DeviceIdType.MESH,
        )
        d.start()
        return d

    def rs_step(t, tx, rx, dst, send_sems, recv_sems):
        d = pltpu.make_async_remote_copy(
            src_ref=tx.at[t],
            dst_ref=rx.at[t],
            send_sem=send_sems.at[t],
            recv_sem=recv_sems.at[t],
            device_id=(dst,),
            device_id_type=pl.DeviceIdType.MESH,
        )
        d.start()
        return d

    barrier_sem = pltpu.get_barrier_semaphore()
    for nbr in (left, right):
        pl.semaphore_signal(barrier_sem, inc=1, device_id=(nbr,),
                            device_id_type=pl.DeviceIdType.MESH)
    pl.semaphore_wait(barrier_sem, 2)

    xg_ref[pl.ds(pl.multiple_of(my * SQ_SHARD, SQ_SHARD), SQ_SHARD), :] = \
        x_ref[:, :]
    agr = [ag_hop(0, my, 0, right, agr_send, agr_recv)]
    agl = [ag_hop(0, my, 1, left, agl_send, agl_recv)]
    compute_pair(my, my)

    rsr, rsl = [], []
    for h in range(N_DEV - 1):
        agr[h].wait_recv()
        cr = (my + N_DEV - 1 - h) % N_DEV
        if h < N_DEV - 2:
            agr.append(ag_hop(h + 1, cr, 0, right, agr_send, agr_recv))
        agl[h].wait_recv()
        cl = (my + 1 + h) % N_DEV
        if h < N_DEV - 2:
            agl.append(ag_hop(h + 1, cl, 1, left, agl_send, agl_recv))

        compute_pair(cr, cl)

        rows_r = pl.ds(pl.multiple_of(cr * SQ_SHARD, HALF), HALF)
        if h == 0:
            acc_r = p_ref[rows_r, :]
        else:
            rsr[h - 1].wait_recv()
            acc_r = p_ref[rows_r, :] + rs_rxr[h - 1, :, :].astype(jnp.float32)
        rs_txr[h, :, :] = acc_r.astype(jnp.bfloat16)
        rsr.append(rs_step(h, rs_txr, rs_rxr, right, rsr_send, rsr_recv))

        rows_l = pl.ds(pl.multiple_of(cl * SQ_SHARD + HALF, HALF), HALF)
        if h == 0:
            acc_l = p_ref[rows_l, :]
        else:
            rsl[h - 1].wait_recv()
            acc_l = p_ref[rows_l, :] + rs_rxl[h - 1, :, :].astype(jnp.float32)
        rs_txl[h, :, :] = acc_l.astype(jnp.bfloat16)
        rsl.append(rs_step(h, rs_txl, rs_rxl, left, rsl_send, rsl_recv))

    rsr[N_DEV - 2].wait_recv()
    out_ref[0, 0:HALF, :] = (
        p_ref[pl.ds(pl.multiple_of(my * SQ_SHARD, HALF), HALF), :]
        + rs_rxr[N_DEV - 2, :, :].astype(jnp.float32))
    rsl[N_DEV - 2].wait_recv()
    out_ref[0, HALF:SQ_SHARD, :] = (
        p_ref[pl.ds(pl.multiple_of(my * SQ_SHARD + HALF, HALF), HALF), :]
        + rs_rxl[N_DEV - 2, :, :].astype(jnp.float32))

    for d in agr + agl + rsr + rsl:
        d.wait_send()


def kernel(x, Wq, K_ext, V_ext, Wo):
    xs = x[0].astype(jnp.bfloat16)
    wq = Wq.astype(jnp.bfloat16)
    wo = Wo.astype(jnp.bfloat16)

    out = pl.pallas_call(
        _attn_body,
        out_shape=jax.ShapeDtypeStruct((1, SQ_SHARD, D_MODEL), jnp.float32),
        in_specs=[
            pl.BlockSpec(memory_space=pltpu.VMEM),
            pl.BlockSpec(memory_space=pltpu.VMEM),
            pl.BlockSpec(memory_space=pltpu.MemorySpace.HBM),
            pl.BlockSpec(memory_space=pltpu.MemorySpace.HBM),
            pl.BlockSpec(memory_space=pltpu.VMEM),
        ],
        out_specs=pl.BlockSpec(memory_space=pltpu.VMEM),
        scratch_shapes=[
            pltpu.VMEM((SQ, D_MODEL), jnp.bfloat16),
            pltpu.VMEM((SQ_SHARD, D_MODEL), jnp.bfloat16),
            pltpu.VMEM((SQ_SHARD, D_MODEL), jnp.bfloat16),
            pltpu.VMEM((SQ, D_MODEL), jnp.float32),
            pltpu.VMEM((HQ_SHARD, KV_USED, DH), jnp.float32),
            pltpu.VMEM((HQ_SHARD, KV_USED, DH), jnp.float32),
            pltpu.VMEM((N_DEV - 1, HALF, D_MODEL), jnp.bfloat16),
            pltpu.VMEM((N_DEV - 1, HALF, D_MODEL), jnp.bfloat16),
            pltpu.VMEM((N_DEV - 1, HALF, D_MODEL), jnp.bfloat16),
            pltpu.VMEM((N_DEV - 1, HALF, D_MODEL), jnp.bfloat16),
            pltpu.SemaphoreType.DMA((2 * HQ_SHARD,)),
            pltpu.SemaphoreType.DMA((N_DEV - 1,)),
            pltpu.SemaphoreType.DMA((N_DEV - 1,)),
            pltpu.SemaphoreType.DMA((N_DEV - 1,)),
            pltpu.SemaphoreType.DMA((N_DEV - 1,)),
            pltpu.SemaphoreType.DMA((N_DEV - 1,)),
            pltpu.SemaphoreType.DMA((N_DEV - 1,)),
            pltpu.SemaphoreType.DMA((N_DEV - 1,)),
            pltpu.SemaphoreType.DMA((N_DEV - 1,)),
        ],
        compiler_params=pltpu.CompilerParams(collective_id=0),
    )(xs, wq, K_ext, V_ext, wo)
    return out


# device time: 34711 ns/iter; 3.0894x vs baseline; 1.0406x over previous
import jax
import jax.numpy as jnp
from jax import lax
from jax.experimental import pallas as pl
from jax.experimental.pallas import tpu as pltpu

N_DEV = 4
SQ_SHARD = 256
HALF = 128
SQ = 1024
D_MODEL = 1024
HQ_SHARD = 8
DH = 128
WINDOW = 128
KV_HWIN = 384
KV_USED = 1152
SCALE = 0.08838834764831843


def _attn_body(x_ref, wq_ref, k_hbm, v_hbm, wo_ref, out_ref,
               xg_ref, xq_ref, ctx_ref, p_ref, k_ref, v_ref,
               rs_txr, rs_rxr, rs_txl, rs_rxl,
               kv_sems,
               agr_send, agr_recv, agl_send, agl_recv,
               rsr_send, rsr_recv, rsl_send, rsl_recv):
    my = lax.axis_index("i")
    left = (my + N_DEV - 1) % N_DEV
    right = (my + 1) % N_DEV

    h0 = my * HQ_SHARD
    kv_dmas = []
    for hh in range(HQ_SHARD):
        for src, dst, slot in ((k_hbm, k_ref, 2 * hh),
                               (v_hbm, v_ref, 2 * hh + 1)):
            d = pltpu.make_async_copy(
                src.at[0, pl.ds(0, KV_USED), h0 + hh, :],
                dst.at[hh],
                kv_sems.at[slot],
            )
            d.start()
            kv_dmas.append(d)
    kv_waited = [False] * HQ_SHARD

    def attn_rows(qc, r0, qb, ws):
        iq = lax.broadcasted_iota(jnp.int32, (HALF, KV_HWIN), 0) + qb
        ik = lax.broadcasted_iota(jnp.int32, (HALF, KV_HWIN), 1) + ws
        bias = jnp.where(jnp.abs(iq - ik) <= WINDOW,
                         0.0, -1e9).astype(jnp.float32)
        for hh in range(HQ_SHARD):
            if not kv_waited[hh]:
                kv_dmas[2 * hh].wait()
                kv_dmas[2 * hh + 1].wait()
                kv_waited[hh] = True
            qh = qc[r0:r0 + HALF, hh * DH:(hh + 1) * DH]
            kh = k_ref[hh, pl.ds(ws, KV_HWIN), :].astype(jnp.bfloat16)
            s = lax.dot_general(
                qh, kh, (((1,), (1,)), ((), ())),
                preferred_element_type=jnp.float32) + bias
            w = jnp.exp(s)
            denom = jnp.sum(w, axis=1, keepdims=True)
            vh = v_ref[hh, pl.ds(ws, KV_HWIN), :].astype(jnp.bfloat16)
            ctx = jnp.dot(w.astype(jnp.bfloat16), vh,
                          preferred_element_type=jnp.float32)
            ctx = ctx * (1.0 / denom)
            ctx_ref[r0:r0 + HALF,
                    hh * DH:(hh + 1) * DH] = ctx.astype(jnp.bfloat16)

    def _win(qb):
        return pl.multiple_of(
            jnp.clip(qb - WINDOW, 0, KV_USED - KV_HWIN), WINDOW)

    def compute_pair(cr, cl):
        qb_r = pl.multiple_of(cr * SQ_SHARD, HALF)
        qb_l = pl.multiple_of(cl * SQ_SHARD + HALF, HALF)
        xq_ref[0:HALF, :] = xg_ref[pl.ds(qb_r, HALF), :]
        xq_ref[HALF:SQ_SHARD, :] = xg_ref[pl.ds(qb_l, HALF), :]
        qc = (jnp.dot(xq_ref[:, :], wq_ref[:, :],
                      preferred_element_type=jnp.float32)
              * SCALE).astype(jnp.bfloat16)
        attn_rows(qc, 0, qb_r, _win(qb_r))
        attn_rows(qc, HALF, qb_l, _win(qb_l))
        return jnp.dot(ctx_ref[:, :], wo_ref[:, :],
                       preferred_element_type=jnp.float32)

    def compute_own_half(half):
        qb = pl.multiple_of(my * SQ_SHARD + half * HALF, HALF)
        qc = (jnp.dot(xg_ref[pl.ds(qb, HALF), :], wq_ref[:, :],
                      preferred_element_type=jnp.float32)
              * SCALE).astype(jnp.bfloat16)
        attn_rows(qc, 0, qb, _win(qb))
        p_ref[half * HALF:(half + 1) * HALF, :] = jnp.dot(
            ctx_ref[0:HALF, :], wo_ref[:, :],
            preferred_element_type=jnp.float32)

    def ag_hop(h, c, half, dst, send_sems, recv_sems):
        rows = pl.ds(pl.multiple_of(c * SQ_SHARD + half * HALF, HALF), HALF)
        d = pltpu.make_async_remote_copy(
            src_ref=xg_ref.at[rows, :],
            dst_ref=xg_ref.at[rows, :],
            send_sem=send_sems.at[h],
            recv_sem=recv_sems.at[h],
            device_id=(dst,),
            device_id_type=pl.DeviceIdType.MESH,
        )
        d.start()
        return d

    def rs_step(t, tx, rx, dst, send_sems, recv_sems):
        d = pltpu.make_async_remote_copy(
            src_ref=tx.at[t],
            dst_ref=rx.at[t],
            send_sem=send_sems.at[t],
            recv_sem=recv_sems.at[t],
            device_id=(dst,),
            device_id_type=pl.DeviceIdType.MESH,
        )
        d.start()
        return d

    barrier_sem = pltpu.get_barrier_semaphore()
    for nbr in (left, right):
        pl.semaphore_signal(barrier_sem, inc=1, device_id=(nbr,),
                            device_id_type=pl.DeviceIdType.MESH)
    pl.semaphore_wait(barrier_sem, 2)

    xg_ref[pl.ds(pl.multiple_of(my * SQ_SHARD, SQ_SHARD), SQ_SHARD), :] = \
        x_ref[:, :]
    agr = [ag_hop(0, my, 0, right, agr_send, agr_recv)]
    agl = [ag_hop(0, my, 1, left, agl_send, agl_recv)]
    compute_own_half(0)

    rsr, rsl = [], []
    for h in range(N_DEV - 1):
        agr[h].wait_recv()
        cr = (my + N_DEV - 1 - h) % N_DEV
        if h < N_DEV - 2:
            agr.append(ag_hop(h + 1, cr, 0, right, agr_send, agr_recv))
        agl[h].wait_recv()
        cl = (my + 1 + h) % N_DEV
        if h < N_DEV - 2:
            agl.append(ag_hop(h + 1, cl, 1, left, agl_send, agl_recv))

        o = compute_pair(cr, cl)

        if h == 0:
            acc_r = o[0:HALF, :]
        else:
            rsr[h - 1].wait_recv()
            acc_r = o[0:HALF, :] + rs_rxr[h - 1, :, :].astype(jnp.float32)
        rs_txr[h, :, :] = acc_r.astype(jnp.bfloat16)
        rsr.append(rs_step(h, rs_txr, rs_rxr, right, rsr_send, rsr_recv))

        if h == 0:
            acc_l = o[HALF:SQ_SHARD, :]
        else:
            rsl[h - 1].wait_recv()
            acc_l = (o[HALF:SQ_SHARD, :]
                     + rs_rxl[h - 1, :, :].astype(jnp.float32))
        rs_txl[h, :, :] = acc_l.astype(jnp.bfloat16)
        rsl.append(rs_step(h, rs_txl, rs_rxl, left, rsl_send, rsl_recv))

    compute_own_half(1)

    rsr[N_DEV - 2].wait_recv()
    out_ref[0, 0:HALF, :] = (
        p_ref[0:HALF, :] + rs_rxr[N_DEV - 2, :, :].astype(jnp.float32))
    rsl[N_DEV - 2].wait_recv()
    out_ref[0, HALF:SQ_SHARD, :] = (
        p_ref[HALF:SQ_SHARD, :] + rs_rxl[N_DEV - 2, :, :].astype(jnp.float32))

    for d in agr + agl + rsr + rsl:
        d.wait_send()


def kernel(x, Wq, K_ext, V_ext, Wo):
    xs = x[0].astype(jnp.bfloat16)
    wq = Wq.astype(jnp.bfloat16)
    wo = Wo.astype(jnp.bfloat16)

    out = pl.pallas_call(
        _attn_body,
        out_shape=jax.ShapeDtypeStruct((1, SQ_SHARD, D_MODEL), jnp.float32),
        in_specs=[
            pl.BlockSpec(memory_space=pltpu.VMEM),
            pl.BlockSpec(memory_space=pltpu.VMEM),
            pl.BlockSpec(memory_space=pltpu.MemorySpace.HBM),
            pl.BlockSpec(memory_space=pltpu.MemorySpace.HBM),
            pl.BlockSpec(memory_space=pltpu.VMEM),
        ],
        out_specs=pl.BlockSpec(memory_space=pltpu.VMEM),
        scratch_shapes=[
            pltpu.VMEM((SQ, D_MODEL), jnp.bfloat16),
            pltpu.VMEM((SQ_SHARD, D_MODEL), jnp.bfloat16),
            pltpu.VMEM((SQ_SHARD, D_MODEL), jnp.bfloat16),
            pltpu.VMEM((SQ_SHARD, D_MODEL), jnp.float32),
            pltpu.VMEM((HQ_SHARD, KV_USED, DH), jnp.float32),
            pltpu.VMEM((HQ_SHARD, KV_USED, DH), jnp.float32),
            pltpu.VMEM((N_DEV - 1, HALF, D_MODEL), jnp.bfloat16),
            pltpu.VMEM((N_DEV - 1, HALF, D_MODEL), jnp.bfloat16),
            pltpu.VMEM((N_DEV - 1, HALF, D_MODEL), jnp.bfloat16),
            pltpu.VMEM((N_DEV - 1, HALF, D_MODEL), jnp.bfloat16),
            pltpu.SemaphoreType.DMA((2 * HQ_SHARD,)),
            pltpu.SemaphoreType.DMA((N_DEV - 1,)),
            pltpu.SemaphoreType.DMA((N_DEV - 1,)),
            pltpu.SemaphoreType.DMA((N_DEV - 1,)),
            pltpu.SemaphoreType.DMA((N_DEV - 1,)),
            pltpu.SemaphoreType.DMA((N_DEV - 1,)),
            pltpu.SemaphoreType.DMA((N_DEV - 1,)),
            pltpu.SemaphoreType.DMA((N_DEV - 1,)),
            pltpu.SemaphoreType.DMA((N_DEV - 1,)),
        ],
        compiler_params=pltpu.CompilerParams(collective_id=0),
    )(xs, wq, K_ext, V_ext, wo)
    return out


# device time: 34703 ns/iter; 3.0901x vs baseline; 1.0002x over previous
import jax
import jax.numpy as jnp
from jax import lax
from jax.experimental import pallas as pl
from jax.experimental.pallas import tpu as pltpu

N_DEV = 4
SQ_SHARD = 256
HALF = 128
SQ = 1024
D_MODEL = 1024
HQ_SHARD = 8
DH = 128
WINDOW = 128
KV_HWIN = 384
KV_USED = 1152
SCALE = 0.08838834764831843


def _attn_body(x_ref, wq_ref, k_hbm, v_hbm, wo_ref, out_ref,
               xg_ref, xq_ref, ctx_ref, p_ref, k_ref, v_ref,
               rs_txr, rs_rxr, rs_txl, rs_rxl,
               kv_sems,
               agr_send, agr_recv, agl_send, agl_recv,
               rsr_send, rsr_recv, rsl_send, rsl_recv):
    my = lax.axis_index("i")
    left = (my + N_DEV - 1) % N_DEV
    right = (my + 1) % N_DEV

    h0 = my * HQ_SHARD
    kv_dmas = []
    for hh in range(HQ_SHARD):
        for src, dst, slot in ((k_hbm, k_ref, 2 * hh),
                               (v_hbm, v_ref, 2 * hh + 1)):
            d = pltpu.make_async_copy(
                src.at[0, pl.ds(0, KV_USED), h0 + hh, :],
                dst.at[hh],
                kv_sems.at[slot],
            )
            d.start()
            kv_dmas.append(d)
    kv_waited = [False] * HQ_SHARD

    def attn_rows(qc, r0, qb, ws):
        iq = lax.broadcasted_iota(jnp.int32, (HALF, KV_HWIN), 0) + qb
        ik = lax.broadcasted_iota(jnp.int32, (HALF, KV_HWIN), 1) + ws
        bias = jnp.where(jnp.abs(iq - ik) <= WINDOW,
                         0.0, -1e9).astype(jnp.float32)
        for hh in range(HQ_SHARD):
            if not kv_waited[hh]:
                kv_dmas[2 * hh].wait()
                kv_dmas[2 * hh + 1].wait()
                kv_waited[hh] = True
            qh = qc[r0:r0 + HALF, hh * DH:(hh + 1) * DH]
            kh = k_ref[hh, pl.ds(ws, KV_HWIN), :].astype(jnp.bfloat16)
            s = lax.dot_general(
                qh, kh, (((1,), (1,)), ((), ())),
                preferred_element_type=jnp.float32) + bias
            w = jnp.exp(s)
            denom = jnp.sum(w, axis=1, keepdims=True)
            vh = v_ref[hh, pl.ds(ws, KV_HWIN), :].astype(jnp.bfloat16)
            ctx = jnp.dot(w.astype(jnp.bfloat16), vh,
                          preferred_element_type=jnp.float32)
            ctx = ctx * (1.0 / denom)
            ctx_ref[r0:r0 + HALF,
                    hh * DH:(hh + 1) * DH] = ctx.astype(jnp.bfloat16)

    def _win(qb):
        return pl.multiple_of(
            jnp.clip(qb - WINDOW, 0, KV_USED - KV_HWIN), WINDOW)

    def compute_pair(cr, cl):
        qb_r = pl.multiple_of(cr * SQ_SHARD, HALF)
        qb_l = pl.multiple_of(cl * SQ_SHARD + HALF, HALF)
        xq_ref[0:HALF, :] = xg_ref[pl.ds(qb_r, HALF), :]
        xq_ref[HALF:SQ_SHARD, :] = xg_ref[pl.ds(qb_l, HALF), :]
        qc = (jnp.dot(xq_ref[:, :], wq_ref[:, :],
                      preferred_element_type=jnp.float32)
              * SCALE).astype(jnp.bfloat16)
        attn_rows(qc, 0, qb_r, _win(qb_r))
        attn_rows(qc, HALF, qb_l, _win(qb_l))
        return jnp.dot(ctx_ref[:, :], wo_ref[:, :],
                       preferred_element_type=jnp.float32)

    def compute_own_half(half):
        qb = pl.multiple_of(my * SQ_SHARD + half * HALF, HALF)
        qc = (jnp.dot(xg_ref[pl.ds(qb, HALF), :], wq_ref[:, :],
                      preferred_element_type=jnp.float32)
              * SCALE).astype(jnp.bfloat16)
        attn_rows(qc, 0, qb, _win(qb))
        p_ref[half * HALF:(half + 1) * HALF, :] = jnp.dot(
            ctx_ref[0:HALF, :], wo_ref[:, :],
            preferred_element_type=jnp.float32)

    def ag_hop(h, c, half, dst, send_sems, recv_sems):
        rows = pl.ds(pl.multiple_of(c * SQ_SHARD + half * HALF, HALF), HALF)
        d = pltpu.make_async_remote_copy(
            src_ref=xg_ref.at[rows, :],
            dst_ref=xg_ref.at[rows, :],
            send_sem=send_sems.at[h],
            recv_sem=recv_sems.at[h],
            device_id=(dst,),
            device_id_type=pl.DeviceIdType.MESH,
        )
        d.start()
        return d

    def rs_step(t, tx, rx, dst, send_sems, recv_sems):
        d = pltpu.make_async_remote_copy(
            src_ref=tx.at[t],
            dst_ref=rx.at[t],
            send_sem=send_sems.at[t],
            recv_sem=recv_sems.at[t],
            device_id=(dst,),
            device_id_type=pl.DeviceIdType.MESH,
        )
        d.start()
        return d

    barrier_sem = pltpu.get_barrier_semaphore()
    for nbr in (left, right):
        pl.semaphore_signal(barrier_sem, inc=1, device_id=(nbr,),
                            device_id_type=pl.DeviceIdType.MESH)
    pl.semaphore_wait(barrier_sem, 2)

    xg_ref[pl.ds(pl.multiple_of(my * SQ_SHARD, SQ_SHARD), SQ_SHARD), :] = \
        x_ref[:, :]
    agr = [ag_hop(0, my, 0, right, agr_send, agr_recv)]
    agl = [ag_hop(0, my, 1, left, agl_send, agl_recv)]
    compute_own_half(0)

    rsr, rsl = [], []
    for h in range(N_DEV - 1):
        agr[h].wait_recv()
        cr = (my + N_DEV - 1 - h) % N_DEV
        if h < N_DEV - 2:
            agr.append(ag_hop(h + 1, cr, 0, right, agr_send, agr_recv))
        agl[h].wait_recv()
        cl = (my + 1 + h) % N_DEV
        if h < N_DEV - 2:
            agl.append(ag_hop(h + 1, cl, 1, left, agl_send, agl_recv))

        o = compute_pair(cr, cl)

        def rs_r():
            if h == 0:
                acc_r = o[0:HALF, :]
            else:
                rsr[h - 1].wait_recv()
                acc_r = (o[0:HALF, :]
                         + rs_rxr[h - 1, :, :].astype(jnp.float32))
            rs_txr[h, :, :] = acc_r.astype(jnp.bfloat16)
            rsr.append(rs_step(h, rs_txr, rs_rxr, right, rsr_send, rsr_recv))

        def rs_l():
            if h == 0:
                acc_l = o[HALF:SQ_SHARD, :]
            else:
                rsl[h - 1].wait_recv()
                acc_l = (o[HALF:SQ_SHARD, :]
                         + rs_rxl[h - 1, :, :].astype(jnp.float32))
            rs_txl[h, :, :] = acc_l.astype(jnp.bfloat16)
            rsl.append(rs_step(h, rs_txl, rs_rxl, left, rsl_send, rsl_recv))

        for f in ((rs_r, rs_l) if h % 2 == 0 else (rs_l, rs_r)):
            f()

    compute_own_half(1)

    rsr[N_DEV - 2].wait_recv()
    out_ref[0, 0:HALF, :] = (
        p_ref[0:HALF, :] + rs_rxr[N_DEV - 2, :, :].astype(jnp.float32))
    rsl[N_DEV - 2].wait_recv()
    out_ref[0, HALF:SQ_SHARD, :] = (
        p_ref[HALF:SQ_SHARD, :] + rs_rxl[N_DEV - 2, :, :].astype(jnp.float32))

    for d in agr + agl + rsr + rsl:
        d.wait_send()


def kernel(x, Wq, K_ext, V_ext, Wo):
    xs = x[0].astype(jnp.bfloat16)
    wq = Wq.astype(jnp.bfloat16)
    wo = Wo.astype(jnp.bfloat16)

    out = pl.pallas_call(
        _attn_body,
        out_shape=jax.ShapeDtypeStruct((1, SQ_SHARD, D_MODEL), jnp.float32),
        in_specs=[
            pl.BlockSpec(memory_space=pltpu.VMEM),
            pl.BlockSpec(memory_space=pltpu.VMEM),
            pl.BlockSpec(memory_space=pltpu.MemorySpace.HBM),
            pl.BlockSpec(memory_space=pltpu.MemorySpace.HBM),
            pl.BlockSpec(memory_space=pltpu.VMEM),
        ],
        out_specs=pl.BlockSpec(memory_space=pltpu.VMEM),
        scratch_shapes=[
            pltpu.VMEM((SQ, D_MODEL), jnp.bfloat16),
            pltpu.VMEM((SQ_SHARD, D_MODEL), jnp.bfloat16),
            pltpu.VMEM((SQ_SHARD, D_MODEL), jnp.bfloat16),
            pltpu.VMEM((SQ_SHARD, D_MODEL), jnp.float32),
            pltpu.VMEM((HQ_SHARD, KV_USED, DH), jnp.float32),
            pltpu.VMEM((HQ_SHARD, KV_USED, DH), jnp.float32),
            pltpu.VMEM((N_DEV - 1, HALF, D_MODEL), jnp.bfloat16),
            pltpu.VMEM((N_DEV - 1, HALF, D_MODEL), jnp.bfloat16),
            pltpu.VMEM((N_DEV - 1, HALF, D_MODEL), jnp.bfloat16),
            pltpu.VMEM((N_DEV - 1, HALF, D_MODEL), jnp.bfloat16),
            pltpu.SemaphoreType.DMA((2 * HQ_SHARD,)),
            pltpu.SemaphoreType.DMA((N_DEV - 1,)),
            pltpu.SemaphoreType.DMA((N_DEV - 1,)),
            pltpu.SemaphoreType.DMA((N_DEV - 1,)),
            pltpu.SemaphoreType.DMA((N_DEV - 1,)),
            pltpu.SemaphoreType.DMA((N_DEV - 1,)),
            pltpu.SemaphoreType.DMA((N_DEV - 1,)),
            pltpu.SemaphoreType.DMA((N_DEV - 1,)),
            pltpu.SemaphoreType.DMA((N_DEV - 1,)),
        ],
        compiler_params=pltpu.CompilerParams(collective_id=0),
    )(xs, wq, K_ext, V_ext, wo)
    return out
